# baseline (speedup 1.0000x reference)
import functools

import jax
import jax.numpy as jnp
from jax.experimental import pallas as pl

LAYER_SPECS = [(128, 256, 2, True), (512, 256, 2, True), (512, 384, 1, False)]
EDGE_TYPES = [
    ("movie", "has_genre", "genre"),
    ("genre", "has_movie", "movie"),
    ("user", "rated_high", "movie"),
    ("movie", "rated_by", "user"),
    ("conversation", "mentions", "movie"),
    ("movie", "mentioned_in", "conversation"),
]


def _mm_kernel(x_ref, w_ref, o_ref):
    o_ref[...] = jnp.dot(x_ref[...], w_ref[...],
                         preferred_element_type=jnp.float32)


@functools.partial(jax.jit, static_argnames=("bm",))
def _matmul(x, w, bm=512):
    m, k = x.shape
    _, n = w.shape
    grid = (pl.cdiv(m, bm),)
    return pl.pallas_call(
        _mm_kernel,
        grid=grid,
        in_specs=[
            pl.BlockSpec((bm, k), lambda i: (i, 0)),
            pl.BlockSpec((k, n), lambda i: (0, 0)),
        ],
        out_specs=pl.BlockSpec((bm, n), lambda i: (i, 0)),
        out_shape=jax.ShapeDtypeStruct((m, n), jnp.float32),
    )(x, w)


def _gat_conv(x_src, x_dst, edge_index, p, heads, concat, n_dst):
    hs = _matmul(x_src, p["W_src"]).reshape(x_src.shape[0], heads, -1)
    a_src = jnp.sum(hs * p["att_src"][None], axis=-1)
    v_dst = (p["W_dst"].reshape(x_dst.shape[1], heads, -1) * p["att_dst"][None]).sum(-1)
    a_dst = x_dst @ v_dst
    src = edge_index[0]
    dst = edge_index[1]
    alpha = jax.nn.leaky_relu(a_src[src] + a_dst[dst], negative_slope=0.2)
    amax = jax.ops.segment_max(alpha, dst, num_segments=n_dst)
    amax = jax.lax.stop_gradient(jnp.where(jnp.isfinite(amax), amax, 0.0))
    ex = jnp.exp(alpha - amax[dst])
    denom = jax.ops.segment_sum(ex, dst, num_segments=n_dst)
    attn = ex / (denom[dst] + 1e-16)
    msg = hs[src] * attn[:, :, None]
    out = jax.ops.segment_sum(msg, dst, num_segments=n_dst)
    if concat:
        out = out.reshape(n_dst, -1)
    else:
        out = out.mean(axis=1)
    return out + p["bias"]


def kernel(x_movie, x_user, x_genre, x_conversation, params, ei_has_genre,
           ei_has_movie, ei_rated_high, ei_rated_by, ei_mentions,
           ei_mentioned_in):
    x = {"movie": x_movie, "user": x_user, "genre": x_genre,
         "conversation": x_conversation}
    ei = {"has_genre": ei_has_genre, "has_movie": ei_has_movie,
          "rated_high": ei_rated_high, "rated_by": ei_rated_by,
          "mentions": ei_mentions, "mentioned_in": ei_mentioned_in}
    for i, (f_in, c, h, concat) in enumerate(LAYER_SPECS):
        out = {t: None for t in x}
        for (s, r, d) in EDGE_TYPES:
            p = params[str(i)][r]
            o = _gat_conv(x[s], x[d], ei[r], p, h, concat, x[d].shape[0])
            out[d] = o if out[d] is None else out[d] + o
        x = out
        if i != len(LAYER_SPECS) - 1:
            x = {t: jax.nn.relu(v) for t, v in x.items()}
    return (x["movie"], x["user"], x["genre"], x["conversation"])


# speedup vs baseline: 8.0410x; 8.0410x over previous
import functools

import jax
import jax.numpy as jnp
from jax import lax
from jax.experimental import pallas as pl
from jax.experimental.pallas import tpu as pltpu
from jax.experimental.pallas import tpu_sc as plsc

N_NODES = {"movie": 10000, "user": 10000, "genre": 500, "conversation": 5000}
RELS = [
    ("has_genre", "movie", "genre", 30000),
    ("has_movie", "genre", "movie", 30000),
    ("rated_high", "user", "movie", 160000),
    ("rated_by", "movie", "user", 160000),
    ("mentions", "conversation", "movie", 25000),
    ("mentioned_in", "movie", "conversation", 25000),
]
LAYERS = [(128, 256, 2, True), (512, 256, 2, True), (512, 384, 1, False)]
TYPES = ["movie", "user", "genre", "conversation"]
GROUPS = [["movie", "genre"], ["user"], ["conversation"]]

NT = 32
CH = 128


def _ru(x, m):
    return -(-x // m) * m


EPAD = {name: _ru(e, CH * NT) for name, _, _, e in RELS}
NPAD = {t: _ru(n + 1, 16) for t, n in N_NODES.items()}
ASIZE = 20032


def _mesh():
    return plsc.VectorSubcoreMesh(core_axis_name="c", subcore_axis_name="s",
                                  num_cores=2, num_subcores=16)


def _den_layout(h):
    bases, off = {}, 0
    for name, _, d, _ in RELS:
        bases[name] = off
        off += _ru(h * NPAD[d], 128)
    return bases, off


def _make_k2(h, suffix):
    dbases, dtot = _den_layout(h)
    out_type = tuple(
        jax.ShapeDtypeStruct((h * EPAD[name],), jnp.float32)
        for name, _, _, _ in RELS
    ) + (jax.ShapeDtypeStruct((NT * dtot,), jnp.float32),)
    scratch = [
        pltpu.VMEM((ASIZE,), jnp.float32),
        pltpu.VMEM((ASIZE,), jnp.float32),
        pltpu.VMEM((ASIZE,), jnp.float32),
        pltpu.VMEM((CH,), jnp.int32),
        pltpu.VMEM((CH,), jnp.int32),
        pltpu.VMEM((2 * CH,), jnp.float32),
    ]

    @functools.partial(pl.kernel, out_type=out_type, mesh=_mesh(),
                       scratch_types=scratch,
                       compiler_params=pltpu.CompilerParams(
                           needs_layout_passes=False),
                       name="gat_edge_softmax_" + suffix)
    def k2(*refs):
        ins = refs[:24]
        exouts = refs[24:30]
        den = refs[30]
        asb, adb, slab, sbuf, dbuf, exbuf = refs[31:]
        wid = lax.axis_index("c") * 16 + lax.axis_index("s")
        for ri, (name, st, dt, _e) in enumerate(RELS):
            srcr, dstr = ins[ri], ins[6 + ri]
            asr, adr = ins[12 + ri], ins[18 + ri]
            ns, nd = N_NODES[st] * h, N_NODES[dt] * h
            npd = NPAD[dt]
            hnp = h * npd
            epad = EPAD[name]
            exout = exouts[ri]
            pltpu.sync_copy(asr, asb.at[pl.ds(0, ns)])
            pltpu.sync_copy(adr, adb.at[pl.ds(0, nd)])

            @pl.loop(0, hnp // 16)
            def _zero(i):
                slab[pl.ds(i * 16, 16)] = jnp.zeros((16,), jnp.float32)

            cpt = epad // CH // NT
            c0 = wid * cpt

            @pl.loop(c0, c0 + cpt)
            def _chunk(c, _srcr=srcr, _dstr=dstr, _exout=exout, _npd=npd,
                       _epad=epad):
                b = c * CH
                pltpu.sync_copy(_srcr.at[pl.ds(b, CH)], sbuf)
                pltpu.sync_copy(_dstr.at[pl.ds(b, CH)], dbuf)
                for j in range(CH // 16):
                    s16 = sbuf[pl.ds(j * 16, 16)]
                    d16 = dbuf[pl.ds(j * 16, 16)]
                    for hh in range(h):
                        av = plsc.load_gather(asb, [s16 * h + hh])
                        bv = plsc.load_gather(adb, [d16 * h + hh])
                        al = av + bv
                        al = jnp.maximum(al, al * 0.2)
                        ex = jnp.exp(al)
                        plsc.addupdate_scatter(slab, [d16 + hh * _npd], ex)
                        exbuf[pl.ds(hh * CH + j * 16, 16)] = ex
                for hh in range(h):
                    pltpu.sync_copy(
                        exbuf.at[pl.ds(hh * CH, CH)],
                        _exout.at[pl.ds(hh * _epad + b, CH)])

            pltpu.sync_copy(
                slab.at[pl.ds(0, hnp)],
                den.at[pl.ds(wid * dtot + dbases[name], hnp)])

    return k2, dtot


def _rdenom(den2d):
    dtot = den2d.shape[1]

    def body(dref, oref):
        s = jnp.sum(dref[...], axis=0, keepdims=True)
        oref[...] = 1.0 / (s + 1e-16)

    return pl.pallas_call(
        body, out_shape=jax.ShapeDtypeStruct((1, dtot), jnp.float32))(den2d)


def _make_k3(h, ch, npass, p, group_rels, tbases, slab_rows, dbases, suffix):
    head = (p * 128) // ch
    nrel = len(group_rels)
    out_type = jax.ShapeDtypeStruct((2 * slab_rows, 128), jnp.float32)
    scratch = [
        pltpu.VMEM_SHARED((slab_rows, 128), jnp.float32),
        pltpu.VMEM((CH, 128), jnp.float32),
        pltpu.VMEM((10016,), jnp.float32),
        pltpu.VMEM((CH,), jnp.int32),
        pltpu.VMEM((CH,), jnp.int32),
        pltpu.VMEM((CH,), jnp.float32),
        pltpu.VMEM((CH,), jnp.int32),
        pltpu.VMEM((CH,), jnp.int32),
        pltpu.VMEM((CH,), jnp.float32),
        pltpu.SemaphoreType.DMA,
    ]

    @functools.partial(pl.kernel, out_type=out_type, mesh=_mesh(),
                       scratch_types=scratch,
                       compiler_params=pltpu.CompilerParams(
                           needs_layout_passes=False),
                       name="gat_aggregate_" + suffix)
    def k3(*refs):
        srcs = refs[0:nrel]
        dsts = refs[nrel:2 * nrel]
        exs = refs[2 * nrel:3 * nrel]
        hss = refs[3 * nrel:4 * nrel]
        rdfl = refs[4 * nrel]
        zz = refs[4 * nrel + 1]
        out = refs[4 * nrel + 2]
        shared, rows, rdb, sbuf, dbuf, ebuf, gix, six, abuf, sem = refs[
            4 * nrel + 3:]
        wid = lax.axis_index("c") * 16 + lax.axis_index("s")
        sid = lax.axis_index("s")
        sc = lax.axis_index("c")
        rpt = slab_rows // 16
        r0 = sid * rpt
        nfull, rem = rpt // CH, rpt % CH

        pltpu.sync_copy(zz, rows)
        if nfull:
            @pl.loop(0, nfull)
            def _zi(i):
                pltpu.sync_copy(rows, shared.at[pl.ds(r0 + i * CH, CH)])
        if rem:
            pltpu.sync_copy(rows.at[pl.ds(0, rem)],
                            shared.at[pl.ds(r0 + nfull * CH, rem)])
        plsc.subcore_barrier()

        for ri, (name, st, dt, _e) in enumerate(group_rels):
            npd = NPAD[dt]
            epad = EPAD[name]
            sb = tbases[dt]
            pltpu.sync_copy(
                rdfl.at[pl.ds(dbases[name] + head * npd, npd)],
                rdb.at[pl.ds(0, npd)])
            cpt = epad // CH // NT
            c0 = wid * cpt

            @pl.loop(0, cpt)
            def _chunk(ci, _src=srcs[ri], _dst=dsts[ri], _ex=exs[ri],
                       _hs=hss[ri], _sb=sb, _epad=epad, _c0=c0):
                b = (_c0 + ci) * CH
                pltpu.sync_copy(_src.at[pl.ds(b, CH)], sbuf)
                pltpu.sync_copy(_dst.at[pl.ds(b, CH)], dbuf)
                pltpu.sync_copy(_ex.at[pl.ds(head * _epad + b, CH)], ebuf)
                for j in range(CH // 16):
                    s16 = sbuf[pl.ds(j * 16, 16)]
                    d16 = dbuf[pl.ds(j * 16, 16)]
                    gix[pl.ds(j * 16, 16)] = s16 * npass + p
                    rdv = plsc.load_gather(rdb, [d16])
                    abuf[pl.ds(j * 16, 16)] = ebuf[pl.ds(j * 16, 16)] * rdv
                    six[pl.ds(j * 16, 16)] = d16 + _sb
                pltpu.async_copy(_hs.at[gix], rows, sem).wait()

                @pl.loop(0, CH)
                def _scale(k):
                    av = plsc.load_gather(abuf, [jnp.broadcast_to(k, (16,))])
                    for j in range(CH // 16):
                        rows[k, pl.ds(j * 16, 16)] = (
                            rows[k, pl.ds(j * 16, 16)] * av)

                pltpu.sync_copy(rows, shared.at[six], add=True)

        plsc.subcore_barrier()
        ob = sc * slab_rows + r0
        if nfull:
            @pl.loop(0, nfull)
            def _wo(i):
                pltpu.sync_copy(shared.at[pl.ds(r0 + i * CH, CH)],
                                out.at[pl.ds(ob + i * CH, CH)])
        if rem:
            pltpu.sync_copy(shared.at[pl.ds(r0 + nfull * CH, rem)],
                            out.at[pl.ds(ob + nfull * CH, rem)])

    return k3


def _proj(x, ws_list, wa, bm=512):
    n, f = x.shape
    nw = len(ws_list)
    grid = (pl.cdiv(n, bm),)

    def body(*refs):
        xr = refs[0]
        wrs = refs[1:1 + nw]
        war = refs[1 + nw]
        outs = refs[2 + nw:2 + 2 * nw]
        oa = refs[2 + 2 * nw]
        xv = xr[...]
        for wr, orf in zip(wrs, outs):
            orf[...] = jnp.dot(xv, wr[...], preferred_element_type=jnp.float32)
        oa[...] = jnp.dot(xv, war[...], preferred_element_type=jnp.float32)

    in_specs = ([pl.BlockSpec((bm, f), lambda i: (i, 0))]
                + [pl.BlockSpec((f, w.shape[1]), lambda i: (0, 0))
                   for w in ws_list]
                + [pl.BlockSpec((f, 128), lambda i: (0, 0))])
    out_specs = ([pl.BlockSpec((bm, w.shape[1]), lambda i: (i, 0))
                  for w in ws_list]
                 + [pl.BlockSpec((bm, 128), lambda i: (i, 0))])
    out_shape = ([jax.ShapeDtypeStruct((n, w.shape[1]), jnp.float32)
                  for w in ws_list]
                 + [jax.ShapeDtypeStruct((n, 128), jnp.float32)])
    return pl.pallas_call(body, grid=grid, in_specs=in_specs,
                          out_specs=out_specs, out_shape=out_shape)(
                              x, *ws_list, wa)


def _combine(parts, bias, n, width, relu, bm=512):
    npass = len(parts)

    def body(*refs):
        ins = refs[:npass]
        br = refs[npass]
        orf = refs[npass + 1]
        for p in range(npass):
            v = ins[p][0] + ins[p][1] + br[0, p * 128:(p + 1) * 128]
            orf[:, p * 128:(p + 1) * 128] = jnp.maximum(v, 0.0) if relu else v

    in_specs = ([pl.BlockSpec((2, bm, 128), lambda i: (0, i, 0))] * npass
                + [pl.BlockSpec((1, width), lambda i: (0, 0))])
    return pl.pallas_call(
        body, grid=(pl.cdiv(n, bm),), in_specs=in_specs,
        out_specs=pl.BlockSpec((bm, width), lambda i: (i, 0)),
        out_shape=jax.ShapeDtypeStruct((n, width), jnp.float32))(
            *parts, bias.reshape(1, width))


def kernel(x_movie, x_user, x_genre, x_conversation, params, ei_has_genre,
           ei_has_movie, ei_rated_high, ei_rated_by, ei_mentions,
           ei_mentioned_in):
    x = {"movie": x_movie, "user": x_user, "genre": x_genre,
         "conversation": x_conversation}
    ei = {"has_genre": ei_has_genre, "has_movie": ei_has_movie,
          "rated_high": ei_rated_high, "rated_by": ei_rated_by,
          "mentions": ei_mentions, "mentioned_in": ei_mentioned_in}

    srcp, dstp = {}, {}
    for name, _s, d, e in RELS:
        pad = EPAD[name] - e
        srcp[name] = jnp.concatenate(
            [ei[name][0], jnp.zeros((pad,), jnp.int32)])
        dstp[name] = jnp.concatenate(
            [ei[name][1], jnp.full((pad,), N_NODES[d], jnp.int32)])

    for l, (f_in, chd, h, concat) in enumerate(LAYERS):
        width = h * chd if concat else chd
        npass = width // 128
        lp = params[str(l)]

        wsrc, vsrc, vdst = {}, {}, {}
        for name, _s, _d, _e in RELS:
            pr = lp[name]
            wsrc[name] = pr["W_src"]
            w3s = pr["W_src"].reshape(f_in, h, chd)
            w3d = pr["W_dst"].reshape(f_in, h, chd)
            vsrc[name] = jnp.einsum("fhc,hc->fh", w3s, pr["att_src"])
            vdst[name] = jnp.einsum("fhc,hc->fh", w3d, pr["att_dst"])

        a_src, a_dst, hs = {}, {}, {}
        for t in TYPES:
            src_rels = [r for r in RELS if r[1] == t]
            dst_rels = [r for r in RELS if r[2] == t]
            ws_list = [wsrc[r[0]] for r in src_rels]
            acols = ([vsrc[r[0]] for r in src_rels]
                     + [vdst[r[0]] for r in dst_rels])
            na = sum(c.shape[1] for c in acols)
            wa = jnp.concatenate(
                acols + [jnp.zeros((f_in, 128 - na), jnp.float32)], axis=1)
            outs = _proj(x[t], ws_list, wa)
            for i, r in enumerate(src_rels):
                hs[r[0]] = outs[i]
            ac = outs[-1]
            off = 0
            for r in src_rels:
                a_src[r[0]] = ac[:, off:off + h].reshape(-1)
                off += h
            for r in dst_rels:
                a_dst[r[0]] = ac[:, off:off + h].reshape(-1)
                off += h

        k2, dtot = _make_k2(h, f"l{l}")
        k2outs = k2(*([srcp[r[0]] for r in RELS] + [dstp[r[0]] for r in RELS]
                      + [a_src[r[0]] for r in RELS]
                      + [a_dst[r[0]] for r in RELS]))
        ex = {r[0]: k2outs[i] for i, r in enumerate(RELS)}
        rden = _rdenom(k2outs[6].reshape(NT, dtot)).reshape(-1)
        dbases, _ = _den_layout(h)

        zz = jnp.zeros((CH, 128), jnp.float32)
        partials = {}
        tbases_all = {}
        for g, gtypes in enumerate(GROUPS):
            rels_g = [r for r in RELS if r[2] in gtypes]
            tbases, off = {}, 0
            for t in gtypes:
                tbases[t] = off
                off += NPAD[t]
            slab_rows = _ru(off, 128)
            tbases_all[g] = tbases
            for p in range(npass):
                k3 = _make_k3(h, chd, npass, p, rels_g, tbases, slab_rows,
                              dbases, f"l{l}g{g}p{p}")
                o = k3(*([srcp[r[0]] for r in rels_g]
                         + [dstp[r[0]] for r in rels_g]
                         + [ex[r[0]] for r in rels_g]
                         + [hs[r[0]].reshape(-1, 128) for r in rels_g]
                         + [rden, zz]))
                partials[(g, p)] = o.reshape(2, slab_rows, 128)

        newx = {}
        for t in TYPES:
            g = next(i for i, gt in enumerate(GROUPS) if t in gt)
            tb = tbases_all[g][t]
            parts = [partials[(g, p)][:, tb:tb + NPAD[t], :]
                     for p in range(npass)]
            bias_tot = sum(lp[r[0]]["bias"] for r in RELS if r[2] == t)
            newx[t] = _combine(parts, bias_tot, N_NODES[t], width,
                               relu=(l < len(LAYERS) - 1))
        x = newx

    return (x["movie"], x["user"], x["genre"], x["conversation"])
